# Initial kernel scaffold; baseline (speedup 1.0000x reference)
import jax
import jax.numpy as jnp
from jax.experimental import pallas as pl


def kernel(features, edge_index, W1, b1, W2, b2):
    raise NotImplementedError("write your pallas kernel here")


# speedup vs baseline: 4.4561x; 4.4561x over previous
import functools

import jax
import jax.numpy as jnp
from jax import lax
from jax.experimental import pallas as pl
from jax.experimental.pallas import tpu as pltpu
from jax.experimental.pallas import tpu_sc as plsc

N = 10000
E = 320000
D = 128
DH = D // 2

NC = 2
NS = 16
CLEN = 80
CHUNKS = E // (NS * 2 * CLEN)
VECS = CLEN // 16
TROWS = 624
TAIL0 = TROWS * NS
TAILN = N - TAIL0

_MESH = plsc.VectorSubcoreMesh(core_axis_name="c", subcore_axis_name="s")


def _unpack_edges(pk_v, src_v, dst_v, cc):
    def body(j, carry):
        for k in range(VECS):
            v = pk_v[cc, j, pl.ds(k * 16, 16)]
            src_v[cc, j, pl.ds(k * 16, 16)] = lax.shift_right_logical(v, 16)
            dst_v[cc, j, pl.ds(k * 16, 16)] = lax.bitwise_and(v, 0xFFFF)
        return carry

    lax.fori_loop(0, CHUNKS, body, 0)


@functools.partial(
    pl.kernel,
    mesh=_MESH,
    compiler_params=pltpu.CompilerParams(use_tc_tiling_on_sc=False),
    out_type=jax.ShapeDtypeStruct((NC, 2, N, 16), jnp.float32),
    scratch_types=[
        pltpu.VMEM((1, CHUNKS, CLEN), jnp.int32),
        pltpu.VMEM((1, CHUNKS, CLEN), jnp.int32),
        pltpu.VMEM((1, CHUNKS, CLEN), jnp.int32),
        pltpu.VMEM((CLEN, 16), jnp.float32),
        pltpu.VMEM_SHARED((N, 16), jnp.float32),
        pltpu.VMEM_SHARED((N, 16), jnp.float32),
    ],
)
def _sc_degrees(pk_hbm, ones_hbm, zeros_hbm, out_hbm,
                pk_v, src_v, dst_v, ones_v, acc_s, acc_d):
    c = lax.axis_index("c")
    s = lax.axis_index("s")
    pltpu.sync_copy(pk_hbm.at[s, c], pk_v.at[0])
    _unpack_edges(pk_v, src_v, dst_v, 0)
    pltpu.sync_copy(ones_hbm, ones_v)
    row0 = s * TROWS

    def zbody(i, carry):
        pltpu.sync_copy(zeros_hbm, acc_s.at[pl.ds(row0 + i * TAILN, TAILN)])
        pltpu.sync_copy(zeros_hbm, acc_d.at[pl.ds(row0 + i * TAILN, TAILN)])
        return carry

    lax.fori_loop(0, TROWS // TAILN, zbody, 0)

    @pl.when(s == 0)
    def _():
        pltpu.sync_copy(zeros_hbm, acc_s.at[pl.ds(TAIL0, TAILN)])
        pltpu.sync_copy(zeros_hbm, acc_d.at[pl.ds(TAIL0, TAILN)])

    plsc.subcore_barrier()

    def body(j, carry):
        pltpu.sync_copy(ones_v, acc_s.at[src_v.at[0, j]], add=True)
        pltpu.sync_copy(ones_v, acc_d.at[dst_v.at[0, j]], add=True)
        return carry

    lax.fori_loop(0, CHUNKS, body, 0)
    plsc.subcore_barrier()
    pltpu.sync_copy(acc_s.at[pl.ds(row0, TROWS)],
                    out_hbm.at[c, 0, pl.ds(row0, TROWS)])
    pltpu.sync_copy(acc_d.at[pl.ds(row0, TROWS)],
                    out_hbm.at[c, 1, pl.ds(row0, TROWS)])

    @pl.when(s == 0)
    def _():
        pltpu.sync_copy(acc_s.at[pl.ds(TAIL0, TAILN)],
                        out_hbm.at[c, 0, pl.ds(TAIL0, TAILN)])
        pltpu.sync_copy(acc_d.at[pl.ds(TAIL0, TAILN)],
                        out_hbm.at[c, 1, pl.ds(TAIL0, TAILN)])


@functools.partial(
    pl.kernel,
    mesh=_MESH,
    compiler_params=pltpu.CompilerParams(use_tc_tiling_on_sc=False),
    out_type=jax.ShapeDtypeStruct((NC, N, DH), jnp.float32),
    scratch_types=[
        pltpu.VMEM((2, CHUNKS, CLEN), jnp.int32),
        pltpu.VMEM((2, CHUNKS, CLEN), jnp.int32),
        pltpu.VMEM((2, CHUNKS, CLEN), jnp.int32),
        pltpu.VMEM((2, CLEN, DH), jnp.float32),
        pltpu.VMEM_SHARED((N, DH), jnp.float32),
        pltpu.SemaphoreType.DMA,
    ],
)
def _sc_aggregate(h_hbm, pk_hbm, zeros_hbm, out_hbm,
                  pk_v, src_v, dst_v, rows_v, acc, gsem):
    c = lax.axis_index("c")
    s = lax.axis_index("s")
    pltpu.sync_copy(pk_hbm.at[s], pk_v)
    _unpack_edges(pk_v, src_v, dst_v, 0)
    _unpack_edges(pk_v, src_v, dst_v, 1)
    row0 = s * TROWS

    def zbody(i, carry):
        pltpu.sync_copy(zeros_hbm, acc.at[pl.ds(row0 + i * TAILN, TAILN)])
        return carry

    lax.fori_loop(0, TROWS // TAILN, zbody, 0)

    @pl.when(s == 0)
    def _():
        pltpu.sync_copy(zeros_hbm, acc.at[pl.ds(TAIL0, TAILN)])

    plsc.subcore_barrier()

    for cc in range(2):
        def body(j, carry):
            pltpu.async_copy(h_hbm.at[c].at[src_v.at[cc, j]],
                             rows_v.at[0], gsem).wait()
            pltpu.sync_copy(rows_v.at[0], acc.at[dst_v.at[cc, j]], add=True)
            return carry

        lax.fori_loop(0, CHUNKS, body, 0)

    plsc.subcore_barrier()
    pltpu.sync_copy(acc.at[pl.ds(row0, TROWS)],
                    out_hbm.at[c, pl.ds(row0, TROWS)])

    @pl.when(s == 0)
    def _():
        pltpu.sync_copy(acc.at[pl.ds(TAIL0, TAILN)],
                        out_hbm.at[c, pl.ds(TAIL0, TAILN)])


_BLK = 2000
_GRID = N // _BLK


def _tc_norms_scale_body(p_ref, x_ref, h_ref, no_ref, ni_ref):
    deg_o = p_ref[0, 0, :, 0:1] + p_ref[1, 0, :, 0:1]
    deg_i = p_ref[0, 1, :, 0:1] + p_ref[1, 1, :, 0:1]
    norm_o = jnp.where(deg_o > 0, lax.rsqrt(jnp.maximum(deg_o, 1.0)), 0.0)
    norm_i = jnp.where(deg_i > 0, lax.rsqrt(jnp.maximum(deg_i, 1.0)), 0.0)
    scaled = x_ref[...] * norm_o
    h_ref[0] = scaled[:, :DH]
    h_ref[1] = scaled[:, DH:]
    no_ref[...] = norm_o
    ni_ref[...] = norm_i


def _tc_norms_scale(p, x):
    return pl.pallas_call(
        _tc_norms_scale_body,
        grid=(_GRID,),
        in_specs=[
            pl.BlockSpec((NC, 2, _BLK, 16), lambda i: (0, 0, i, 0)),
            pl.BlockSpec((_BLK, D), lambda i: (i, 0)),
        ],
        out_specs=[
            pl.BlockSpec((2, _BLK, DH), lambda i: (0, i, 0)),
            pl.BlockSpec((_BLK, 1), lambda i: (i, 0)),
            pl.BlockSpec((_BLK, 1), lambda i: (i, 0)),
        ],
        out_shape=[
            jax.ShapeDtypeStruct((2, N, DH), jnp.float32),
            jax.ShapeDtypeStruct((N, 1), jnp.float32),
            jax.ShapeDtypeStruct((N, 1), jnp.float32),
        ],
    )(p, x)


def _tc_dense_body(relu_rescale, q_ref, ni_ref, no_ref, w_ref, b_ref, o_ref):
    t = jnp.concatenate([q_ref[0], q_ref[1]], axis=1) * ni_ref[...]
    r = jnp.dot(t, w_ref[...], preferred_element_type=jnp.float32) + b_ref[...]
    if relu_rescale:
        r = jnp.maximum(r, 0.0) * no_ref[...]
        o_ref[0] = r[:, :DH]
        o_ref[1] = r[:, DH:]
    else:
        o_ref[...] = r


def _tc_dense(q, norm_in, norm_out, w, b, relu_rescale):
    if relu_rescale:
        out_spec = pl.BlockSpec((2, _BLK, DH), lambda i: (0, i, 0))
        out_shape = jax.ShapeDtypeStruct((2, N, DH), jnp.float32)
    else:
        out_spec = pl.BlockSpec((_BLK, D), lambda i: (i, 0))
        out_shape = jax.ShapeDtypeStruct((N, D), jnp.float32)
    return pl.pallas_call(
        functools.partial(_tc_dense_body, relu_rescale),
        grid=(_GRID,),
        in_specs=[
            pl.BlockSpec((NC, _BLK, DH), lambda i: (0, i, 0)),
            pl.BlockSpec((_BLK, 1), lambda i: (i, 0)),
            pl.BlockSpec((_BLK, 1), lambda i: (i, 0)),
            pl.BlockSpec((D, D), lambda i: (0, 0)),
            pl.BlockSpec((1, D), lambda i: (0, 0)),
        ],
        out_specs=out_spec,
        out_shape=out_shape,
    )(q, norm_in, norm_out, w, b)


def kernel(features, edge_index, W1, b1, W2, b2):
    packed = jnp.bitwise_or(
        jnp.left_shift(edge_index[0], 16), edge_index[1]
    ).reshape(NS, 2, CHUNKS, CLEN)
    ones16 = jnp.ones((CLEN, 16), jnp.float32)
    zeros16 = jnp.zeros((TAILN, 16), jnp.float32)
    zerosD = jnp.zeros((TAILN, DH), jnp.float32)
    b1r = b1.reshape(1, D)
    b2r = b2.reshape(1, D)

    deg_p = _sc_degrees(packed, ones16, zeros16)
    h1s, norm_o, norm_i = _tc_norms_scale(deg_p, features)
    q1 = _sc_aggregate(h1s, packed, zerosD)
    h2s = _tc_dense(q1, norm_i, norm_o, W1, b1r, True)
    q2 = _sc_aggregate(h2s, packed, zerosD)
    out = _tc_dense(q2, norm_i, norm_o, W2, b2r, False)
    return out


# speedup vs baseline: 6.8523x; 1.5377x over previous
import functools

import jax
import jax.numpy as jnp
from jax import lax
from jax.experimental import pallas as pl
from jax.experimental.pallas import tpu as pltpu
from jax.experimental.pallas import tpu_sc as plsc

N = 10000
E = 320000
D = 128
DH = D // 2

NC = 2
NS = 16
CLEN = 80
FCHUNKS = E // (NS * CLEN)
DCHUNKS = FCHUNKS // 2
VECS = CLEN // 16
TROWS = 624
TAIL0 = TROWS * NS
TAILN = N - TAIL0

_MESH = plsc.VectorSubcoreMesh(core_axis_name="c", subcore_axis_name="s")


def _unpack_edges(pk_v, src_v, dst_v, nchunks):
    def body(j, carry):
        for k in range(VECS):
            v = pk_v[j, pl.ds(k * 16, 16)]
            src_v[j, pl.ds(k * 16, 16)] = lax.shift_right_logical(v, 16)
            dst_v[j, pl.ds(k * 16, 16)] = lax.bitwise_and(v, 0xFFFF)
        return carry

    lax.fori_loop(0, nchunks, body, 0)


@functools.partial(
    pl.kernel,
    mesh=_MESH,
    compiler_params=pltpu.CompilerParams(use_tc_tiling_on_sc=False),
    out_type=jax.ShapeDtypeStruct((NC, 2, N, 16), jnp.float32),
    scratch_types=[
        pltpu.VMEM((DCHUNKS, CLEN), jnp.int32),
        pltpu.VMEM((DCHUNKS, CLEN), jnp.int32),
        pltpu.VMEM((DCHUNKS, CLEN), jnp.int32),
        pltpu.VMEM((CLEN, 16), jnp.float32),
        pltpu.VMEM_SHARED((N, 16), jnp.float32),
        pltpu.VMEM_SHARED((N, 16), jnp.float32),
        pltpu.SemaphoreType.DMA,
    ],
)
def _sc_degrees(pk_hbm, ones_hbm, zeros_hbm, out_hbm,
                pk_v, src_v, dst_v, ones_v, acc_s, acc_d, ssem):
    c = lax.axis_index("c")
    s = lax.axis_index("s")
    pltpu.sync_copy(pk_hbm.at[s, c], pk_v)
    _unpack_edges(pk_v, src_v, dst_v, DCHUNKS)
    pltpu.sync_copy(ones_hbm, ones_v)
    row0 = s * TROWS
    pltpu.sync_copy(zeros_hbm, acc_s.at[pl.ds(row0, TROWS)])
    pltpu.sync_copy(zeros_hbm, acc_d.at[pl.ds(row0, TROWS)])

    @pl.when(s == 0)
    def _():
        pltpu.sync_copy(zeros_hbm.at[pl.ds(0, TAILN)],
                        acc_s.at[pl.ds(TAIL0, TAILN)])
        pltpu.sync_copy(zeros_hbm.at[pl.ds(0, TAILN)],
                        acc_d.at[pl.ds(TAIL0, TAILN)])

    plsc.subcore_barrier()

    def body(j, carry):
        pltpu.async_copy(ones_v, acc_s.at[src_v.at[j]], ssem, add=True)
        pltpu.async_copy(ones_v, acc_d.at[dst_v.at[j]], ssem, add=True)
        return carry

    lax.fori_loop(0, DCHUNKS, body, 0)

    def drain(j, carry):
        pltpu.make_async_copy(ones_v, acc_s.at[src_v.at[j]], ssem).wait()
        pltpu.make_async_copy(ones_v, acc_d.at[dst_v.at[j]], ssem).wait()
        return carry

    lax.fori_loop(0, DCHUNKS, drain, 0)
    plsc.subcore_barrier()
    pltpu.sync_copy(acc_s.at[pl.ds(row0, TROWS)],
                    out_hbm.at[c, 0, pl.ds(row0, TROWS)])
    pltpu.sync_copy(acc_d.at[pl.ds(row0, TROWS)],
                    out_hbm.at[c, 1, pl.ds(row0, TROWS)])

    @pl.when(s == 0)
    def _():
        pltpu.sync_copy(acc_s.at[pl.ds(TAIL0, TAILN)],
                        out_hbm.at[c, 0, pl.ds(TAIL0, TAILN)])
        pltpu.sync_copy(acc_d.at[pl.ds(TAIL0, TAILN)],
                        out_hbm.at[c, 1, pl.ds(TAIL0, TAILN)])


@functools.partial(
    pl.kernel,
    mesh=_MESH,
    compiler_params=pltpu.CompilerParams(use_tc_tiling_on_sc=False),
    out_type=jax.ShapeDtypeStruct((NC, N, DH), jnp.float32),
    scratch_types=[
        pltpu.VMEM((FCHUNKS, CLEN), jnp.int32),
        pltpu.VMEM((FCHUNKS, CLEN), jnp.int32),
        pltpu.VMEM((FCHUNKS, CLEN), jnp.int32),
        pltpu.VMEM((2, CLEN, DH), jnp.float32),
        pltpu.VMEM_SHARED((N, DH), jnp.float32),
        pltpu.SemaphoreType.DMA,
    ],
)
def _sc_aggregate(h_hbm, pk_hbm, zeros_hbm, out_hbm,
                  pk_v, src_v, dst_v, rows_v, acc, gsem):
    c = lax.axis_index("c")
    s = lax.axis_index("s")
    pltpu.sync_copy(pk_hbm.at[s], pk_v)
    _unpack_edges(pk_v, src_v, dst_v, FCHUNKS)
    row0 = s * TROWS
    pltpu.sync_copy(zeros_hbm, acc.at[pl.ds(row0, TROWS)])

    @pl.when(s == 0)
    def _():
        pltpu.sync_copy(zeros_hbm.at[pl.ds(0, TAILN)],
                        acc.at[pl.ds(TAIL0, TAILN)])

    plsc.subcore_barrier()

    hbm = h_hbm.at[c]
    pltpu.async_copy(hbm.at[src_v.at[0]], rows_v.at[0], gsem)

    def body(j, carry):
        for u in range(2):
            f = 2 * j + u
            pltpu.make_async_copy(hbm.at[src_v.at[f]],
                                  rows_v.at[u], gsem).wait()
            if u == 0:
                pltpu.async_copy(hbm.at[src_v.at[f + 1]], rows_v.at[1], gsem)
            else:
                @pl.when(j < FCHUNKS // 2 - 1)
                def _():
                    pltpu.async_copy(hbm.at[src_v.at[f + 1]],
                                     rows_v.at[0], gsem)
            pltpu.sync_copy(rows_v.at[u], acc.at[dst_v.at[f]], add=True)
        return carry

    lax.fori_loop(0, FCHUNKS // 2, body, 0)

    plsc.subcore_barrier()
    pltpu.sync_copy(acc.at[pl.ds(row0, TROWS)],
                    out_hbm.at[c, pl.ds(row0, TROWS)])

    @pl.when(s == 0)
    def _():
        pltpu.sync_copy(acc.at[pl.ds(TAIL0, TAILN)],
                        out_hbm.at[c, pl.ds(TAIL0, TAILN)])


_BLK = 2000
_GRID = N // _BLK


def _tc_norms_scale_body(p_ref, x_ref, h_ref, no_ref, ni_ref):
    deg_o = p_ref[0, 0, :, 0:1] + p_ref[1, 0, :, 0:1]
    deg_i = p_ref[0, 1, :, 0:1] + p_ref[1, 1, :, 0:1]
    norm_o = jnp.where(deg_o > 0, lax.rsqrt(jnp.maximum(deg_o, 1.0)), 0.0)
    norm_i = jnp.where(deg_i > 0, lax.rsqrt(jnp.maximum(deg_i, 1.0)), 0.0)
    scaled = x_ref[...] * norm_o
    h_ref[0] = scaled[:, :DH]
    h_ref[1] = scaled[:, DH:]
    no_ref[...] = norm_o
    ni_ref[...] = norm_i


def _tc_norms_scale(p, x):
    return pl.pallas_call(
        _tc_norms_scale_body,
        grid=(_GRID,),
        in_specs=[
            pl.BlockSpec((NC, 2, _BLK, 16), lambda i: (0, 0, i, 0)),
            pl.BlockSpec((_BLK, D), lambda i: (i, 0)),
        ],
        out_specs=[
            pl.BlockSpec((2, _BLK, DH), lambda i: (0, i, 0)),
            pl.BlockSpec((_BLK, 1), lambda i: (i, 0)),
            pl.BlockSpec((_BLK, 1), lambda i: (i, 0)),
        ],
        out_shape=[
            jax.ShapeDtypeStruct((2, N, DH), jnp.float32),
            jax.ShapeDtypeStruct((N, 1), jnp.float32),
            jax.ShapeDtypeStruct((N, 1), jnp.float32),
        ],
    )(p, x)


def _tc_dense_body(relu_rescale, q_ref, ni_ref, no_ref, w_ref, b_ref, o_ref):
    t = jnp.concatenate([q_ref[0], q_ref[1]], axis=1) * ni_ref[...]
    r = jnp.dot(t, w_ref[...], preferred_element_type=jnp.float32) + b_ref[...]
    if relu_rescale:
        r = jnp.maximum(r, 0.0) * no_ref[...]
        o_ref[0] = r[:, :DH]
        o_ref[1] = r[:, DH:]
    else:
        o_ref[...] = r


def _tc_dense(q, norm_in, norm_out, w, b, relu_rescale):
    if relu_rescale:
        out_spec = pl.BlockSpec((2, _BLK, DH), lambda i: (0, i, 0))
        out_shape = jax.ShapeDtypeStruct((2, N, DH), jnp.float32)
    else:
        out_spec = pl.BlockSpec((_BLK, D), lambda i: (i, 0))
        out_shape = jax.ShapeDtypeStruct((N, D), jnp.float32)
    return pl.pallas_call(
        functools.partial(_tc_dense_body, relu_rescale),
        grid=(_GRID,),
        in_specs=[
            pl.BlockSpec((NC, _BLK, DH), lambda i: (0, i, 0)),
            pl.BlockSpec((_BLK, 1), lambda i: (i, 0)),
            pl.BlockSpec((_BLK, 1), lambda i: (i, 0)),
            pl.BlockSpec((D, D), lambda i: (0, 0)),
            pl.BlockSpec((1, D), lambda i: (0, 0)),
        ],
        out_specs=out_spec,
        out_shape=out_shape,
    )(q, norm_in, norm_out, w, b)


def kernel(features, edge_index, W1, b1, W2, b2):
    packed = jnp.bitwise_or(
        jnp.left_shift(edge_index[0], 16), edge_index[1]
    )
    pk_deg = packed.reshape(NS, 2, DCHUNKS, CLEN)
    pk_agg = packed.reshape(NS, FCHUNKS, CLEN)
    ones16 = jnp.ones((CLEN, 16), jnp.float32)
    zeros16 = jnp.zeros((TROWS, 16), jnp.float32)
    zerosD = jnp.zeros((TROWS, DH), jnp.float32)
    b1r = b1.reshape(1, D)
    b2r = b2.reshape(1, D)

    deg_p = _sc_degrees(pk_deg, ones16, zeros16)
    h1s, norm_o, norm_i = _tc_norms_scale(deg_p, features)
    q1 = _sc_aggregate(h1s, pk_agg, zerosD)
    h2s = _tc_dense(q1, norm_i, norm_o, W1, b1r, True)
    q2 = _sc_aggregate(h2s, pk_agg, zerosD)
    out = _tc_dense(q2, norm_i, norm_o, W2, b2r, False)
    return out


# speedup vs baseline: 9.9966x; 1.4589x over previous
import functools

import jax
import jax.numpy as jnp
from jax import lax
from jax.experimental import pallas as pl
from jax.experimental.pallas import tpu as pltpu
from jax.experimental.pallas import tpu_sc as plsc

N = 10000
E = 320000
D = 128
DH = D // 2

NC = 2
NS = 16
CLEN = 80
FCHUNKS = E // (NS * CLEN)
DCHUNKS = FCHUNKS // 2
VECS = CLEN // 16
TROWS = 624
TAIL0 = TROWS * NS
TAILN = N - TAIL0

_MESH = plsc.VectorSubcoreMesh(core_axis_name="c", subcore_axis_name="s")


def _unpack_edges(pk_v, src_v, dst_v, nchunks):
    def body(j, carry):
        for k in range(VECS):
            v = pk_v[j, pl.ds(k * 16, 16)]
            src_v[j, pl.ds(k * 16, 16)] = lax.shift_right_logical(v, 16)
            dst_v[j, pl.ds(k * 16, 16)] = lax.bitwise_and(v, 0xFFFF)
        return carry

    lax.fori_loop(0, nchunks, body, 0)


@functools.partial(
    pl.kernel,
    mesh=_MESH,
    compiler_params=pltpu.CompilerParams(use_tc_tiling_on_sc=False),
    out_type=jax.ShapeDtypeStruct((NC, 2, N, 16), jnp.float32),
    scratch_types=[
        pltpu.VMEM((DCHUNKS, CLEN), jnp.int32),
        pltpu.VMEM((DCHUNKS, CLEN), jnp.int32),
        pltpu.VMEM((DCHUNKS, CLEN), jnp.int32),
        pltpu.VMEM((CLEN, 16), jnp.float32),
        pltpu.VMEM_SHARED((N, 16), jnp.float32),
        pltpu.VMEM_SHARED((N, 16), jnp.float32),
        pltpu.SemaphoreType.DMA,
    ],
)
def _sc_degrees(pk_hbm, ones_hbm, zeros_hbm, out_hbm,
                pk_v, src_v, dst_v, ones_v, acc_s, acc_d, ssem):
    c = lax.axis_index("c")
    s = lax.axis_index("s")
    pltpu.sync_copy(pk_hbm.at[s, c], pk_v)
    _unpack_edges(pk_v, src_v, dst_v, DCHUNKS)
    pltpu.sync_copy(ones_hbm, ones_v)
    row0 = s * TROWS
    pltpu.sync_copy(zeros_hbm, acc_s.at[pl.ds(row0, TROWS)])
    pltpu.sync_copy(zeros_hbm, acc_d.at[pl.ds(row0, TROWS)])

    @pl.when(s == 0)
    def _():
        pltpu.sync_copy(zeros_hbm.at[pl.ds(0, TAILN)],
                        acc_s.at[pl.ds(TAIL0, TAILN)])
        pltpu.sync_copy(zeros_hbm.at[pl.ds(0, TAILN)],
                        acc_d.at[pl.ds(TAIL0, TAILN)])

    plsc.subcore_barrier()

    def body(j, carry):
        pltpu.async_copy(ones_v, acc_s.at[src_v.at[j]], ssem, add=True)
        pltpu.async_copy(ones_v, acc_d.at[dst_v.at[j]], ssem, add=True)
        return carry

    lax.fori_loop(0, DCHUNKS, body, 0)

    def drain(j, carry):
        pltpu.make_async_copy(ones_v, acc_s.at[src_v.at[j]], ssem).wait()
        pltpu.make_async_copy(ones_v, acc_d.at[dst_v.at[j]], ssem).wait()
        return carry

    lax.fori_loop(0, DCHUNKS, drain, 0)
    plsc.subcore_barrier()
    pltpu.sync_copy(acc_s.at[pl.ds(row0, TROWS)],
                    out_hbm.at[c, 0, pl.ds(row0, TROWS)])
    pltpu.sync_copy(acc_d.at[pl.ds(row0, TROWS)],
                    out_hbm.at[c, 1, pl.ds(row0, TROWS)])

    @pl.when(s == 0)
    def _():
        pltpu.sync_copy(acc_s.at[pl.ds(TAIL0, TAILN)],
                        out_hbm.at[c, 0, pl.ds(TAIL0, TAILN)])
        pltpu.sync_copy(acc_d.at[pl.ds(TAIL0, TAILN)],
                        out_hbm.at[c, 1, pl.ds(TAIL0, TAILN)])


@functools.partial(
    pl.kernel,
    mesh=_MESH,
    compiler_params=pltpu.CompilerParams(use_tc_tiling_on_sc=False),
    out_type=jax.ShapeDtypeStruct((NC, N, DH), jnp.float32),
    scratch_types=[
        pltpu.VMEM((FCHUNKS, CLEN), jnp.int32),
        pltpu.VMEM((FCHUNKS, CLEN), jnp.int32),
        pltpu.VMEM((FCHUNKS, CLEN), jnp.int32),
        pltpu.VMEM((4, CLEN, DH), jnp.float32),
        pltpu.VMEM_SHARED((N, DH), jnp.float32),
        pltpu.SemaphoreType.DMA,
        pltpu.SemaphoreType.DMA,
    ],
)
def _sc_aggregate(h_hbm, pk_hbm, zeros_hbm, out_hbm,
                  pk_v, src_v, dst_v, rows_v, acc, gsem, ssem):
    c = lax.axis_index("c")
    s = lax.axis_index("s")
    pltpu.sync_copy(pk_hbm.at[s], pk_v)
    _unpack_edges(pk_v, src_v, dst_v, FCHUNKS)
    row0 = s * TROWS
    pltpu.sync_copy(zeros_hbm, acc.at[pl.ds(row0, TROWS)])

    @pl.when(s == 0)
    def _():
        pltpu.sync_copy(zeros_hbm.at[pl.ds(0, TAILN)],
                        acc.at[pl.ds(TAIL0, TAILN)])

    plsc.subcore_barrier()

    hbm = h_hbm.at[c]

    def gather(f, b):
        pltpu.async_copy(hbm.at[src_v.at[f]], rows_v.at[b], gsem)

    def gather_wait(f, b):
        pltpu.make_async_copy(hbm.at[src_v.at[f]], rows_v.at[b], gsem).wait()

    def scatter(f, b):
        pltpu.async_copy(rows_v.at[b], acc.at[dst_v.at[f]], ssem, add=True)

    def scatter_wait(f, b):
        pltpu.make_async_copy(rows_v.at[b], acc.at[dst_v.at[f]], ssem).wait()

    gather(0, 0)
    gather(1, 1)

    def body(j, carry):
        for u in range(4):
            f = 4 * j + u
            gather_wait(f, u)
            scatter(f, u)
            if u < 2:
                @pl.when(j >= 1)
                def _():
                    scatter_wait(f - 2, (u + 2) % 4)
            else:
                scatter_wait(f - 2, (u + 2) % 4)
            gather(f + 2, (u + 2) % 4)
        return carry

    lax.fori_loop(0, (FCHUNKS - 2) // 4, body, 0)
    gather_wait(FCHUNKS - 2, 0)
    scatter(FCHUNKS - 2, 0)
    scatter_wait(FCHUNKS - 4, 2)
    gather_wait(FCHUNKS - 1, 1)
    scatter(FCHUNKS - 1, 1)
    scatter_wait(FCHUNKS - 3, 3)
    scatter_wait(FCHUNKS - 2, 0)
    scatter_wait(FCHUNKS - 1, 1)

    plsc.subcore_barrier()
    pltpu.sync_copy(acc.at[pl.ds(row0, TROWS)],
                    out_hbm.at[c, pl.ds(row0, TROWS)])

    @pl.when(s == 0)
    def _():
        pltpu.sync_copy(acc.at[pl.ds(TAIL0, TAILN)],
                        out_hbm.at[c, pl.ds(TAIL0, TAILN)])


_BLK = 2000
_GRID = N // _BLK


def _tc_norms_scale_body(p_ref, x_ref, h_ref, no_ref, ni_ref):
    deg_o = p_ref[0, 0, :, 0:1] + p_ref[1, 0, :, 0:1]
    deg_i = p_ref[0, 1, :, 0:1] + p_ref[1, 1, :, 0:1]
    norm_o = jnp.where(deg_o > 0, lax.rsqrt(jnp.maximum(deg_o, 1.0)), 0.0)
    norm_i = jnp.where(deg_i > 0, lax.rsqrt(jnp.maximum(deg_i, 1.0)), 0.0)
    scaled = x_ref[...] * norm_o
    h_ref[0] = scaled[:, :DH]
    h_ref[1] = scaled[:, DH:]
    no_ref[...] = norm_o
    ni_ref[...] = norm_i


def _tc_norms_scale(p, x):
    return pl.pallas_call(
        _tc_norms_scale_body,
        grid=(_GRID,),
        in_specs=[
            pl.BlockSpec((NC, 2, _BLK, 16), lambda i: (0, 0, i, 0)),
            pl.BlockSpec((_BLK, D), lambda i: (i, 0)),
        ],
        out_specs=[
            pl.BlockSpec((2, _BLK, DH), lambda i: (0, i, 0)),
            pl.BlockSpec((_BLK, 1), lambda i: (i, 0)),
            pl.BlockSpec((_BLK, 1), lambda i: (i, 0)),
        ],
        out_shape=[
            jax.ShapeDtypeStruct((2, N, DH), jnp.float32),
            jax.ShapeDtypeStruct((N, 1), jnp.float32),
            jax.ShapeDtypeStruct((N, 1), jnp.float32),
        ],
    )(p, x)


def _tc_matmul_body(x_ref, w_ref, o_ref):
    o_ref[...] = jnp.dot(x_ref[...], w_ref[...],
                         preferred_element_type=jnp.float32)


def _tc_matmul(x, w):
    return pl.pallas_call(
        _tc_matmul_body,
        grid=(_GRID,),
        in_specs=[
            pl.BlockSpec((_BLK, D), lambda i: (i, 0)),
            pl.BlockSpec((D, D), lambda i: (0, 0)),
        ],
        out_specs=pl.BlockSpec((_BLK, D), lambda i: (i, 0)),
        out_shape=jax.ShapeDtypeStruct((N, D), jnp.float32),
    )(x, w)


def _tc_mid_body(q_ref, ni_ref, no_ref, w_ref, b_ref, o_ref):
    t = jnp.concatenate([q_ref[0], q_ref[1]], axis=1) * ni_ref[...] + b_ref[...]
    h = jnp.maximum(t, 0.0) * no_ref[...]
    r = jnp.dot(h, w_ref[...], preferred_element_type=jnp.float32)
    o_ref[0] = r[:, :DH]
    o_ref[1] = r[:, DH:]


def _tc_mid(q, norm_in, norm_out, w2, b1):
    return pl.pallas_call(
        _tc_mid_body,
        grid=(_GRID,),
        in_specs=[
            pl.BlockSpec((NC, _BLK, DH), lambda i: (0, i, 0)),
            pl.BlockSpec((_BLK, 1), lambda i: (i, 0)),
            pl.BlockSpec((_BLK, 1), lambda i: (i, 0)),
            pl.BlockSpec((D, D), lambda i: (0, 0)),
            pl.BlockSpec((1, D), lambda i: (0, 0)),
        ],
        out_specs=pl.BlockSpec((2, _BLK, DH), lambda i: (0, i, 0)),
        out_shape=jax.ShapeDtypeStruct((2, N, DH), jnp.float32),
    )(q, norm_in, norm_out, w2, b1)


def _tc_final_body(q_ref, ni_ref, b_ref, o_ref):
    o_ref[...] = (jnp.concatenate([q_ref[0], q_ref[1]], axis=1) * ni_ref[...]
                  + b_ref[...])


def _tc_final(q, norm_in, b2):
    return pl.pallas_call(
        _tc_final_body,
        grid=(_GRID,),
        in_specs=[
            pl.BlockSpec((NC, _BLK, DH), lambda i: (0, i, 0)),
            pl.BlockSpec((_BLK, 1), lambda i: (i, 0)),
            pl.BlockSpec((1, D), lambda i: (0, 0)),
        ],
        out_specs=pl.BlockSpec((_BLK, D), lambda i: (i, 0)),
        out_shape=jax.ShapeDtypeStruct((N, D), jnp.float32),
    )(q, norm_in, b2)


def kernel(features, edge_index, W1, b1, W2, b2):
    packed = jnp.bitwise_or(
        jnp.left_shift(edge_index[0], 16), edge_index[1]
    )
    pk_deg = packed.reshape(NS, 2, DCHUNKS, CLEN)
    pk_agg = packed.reshape(NS, FCHUNKS, CLEN)
    ones16 = jnp.ones((CLEN, 16), jnp.float32)
    zeros16 = jnp.zeros((TROWS, 16), jnp.float32)
    zerosD = jnp.zeros((TROWS, DH), jnp.float32)
    b1r = b1.reshape(1, D)
    b2r = b2.reshape(1, D)

    deg_p = _sc_degrees(pk_deg, ones16, zeros16)
    xw1 = _tc_matmul(features, W1)
    h1s, norm_o, norm_i = _tc_norms_scale(deg_p, xw1)
    q1 = _sc_aggregate(h1s, pk_agg, zerosD)
    h2s = _tc_mid(q1, norm_i, norm_o, W2, b1r)
    q2 = _sc_aggregate(h2s, pk_agg, zerosD)
    out = _tc_final(q2, norm_i, b2r)
    return out
